# baseline (speedup 1.0000x reference)
import functools

import jax
import jax.numpy as jnp
from jax import lax
from jax.experimental import pallas as pl
from jax.experimental.pallas import tpu as pltpu
from jax.experimental.pallas import tpu_sc as plsc

N = 10000
D = 128
H = 128
C = 40
E = 320000

NCORES = 2
NSUB = 16
NW = NCORES * NSUB
CHUNK = 128
NJ = 80
E_PAD = NW * CHUNK * NJ
NCH = E_PAD // CHUNK
NJ0 = 64
NJ1 = 96
NP = 10240
NPT = NP // NSUB
C_PAD = 128

_MESH = plsc.VectorSubcoreMesh(core_axis_name="c", subcore_axis_name="s")


@functools.partial(
    pl.kernel,
    out_type=jax.ShapeDtypeStruct((NCORES, NP, 16), jnp.float32),
    mesh=_MESH,
    scratch_types=[
        pltpu.VMEM((NJ, CHUNK), jnp.int32),
        pltpu.VMEM((CHUNK, 16), jnp.float32),
        pltpu.VMEM((CHUNK, 16), jnp.float32),
        pltpu.VMEM_SHARED((NP, 16), jnp.float32),
    ],
)
def _hist_kernel(col_hbm, hist_hbm, col_v, ones_v, zeros_v, hist_sh):
    c = lax.axis_index("c")
    s = lax.axis_index("s")
    t = c * NSUB + s

    def fill(i, _):
        ones_v[i, :] = jnp.ones((16,), jnp.float32)
        zeros_v[i, :] = jnp.zeros((16,), jnp.float32)
        return 0

    lax.fori_loop(0, CHUNK, fill, 0)

    def zc(k, _):
        pltpu.sync_copy(zeros_v, hist_sh.at[pl.ds(s * NPT + k * CHUNK, CHUNK)])
        return 0

    lax.fori_loop(0, NPT // CHUNK, zc, 0)
    pltpu.sync_copy(col_hbm.at[t], col_v)
    plsc.subcore_barrier()

    def body(j, _):
        pltpu.sync_copy(ones_v, hist_sh.at[col_v.at[j]], add=True)
        return 0

    lax.fori_loop(0, NJ, body, 0)
    plsc.subcore_barrier()
    pltpu.sync_copy(hist_sh.at[pl.ds(s * NPT, NPT)],
                    hist_hbm.at[c, pl.ds(s * NPT, NPT)])


def _make_agg(Wd, tc_tiling=True):
    @functools.partial(
        pl.kernel,
        out_type=jax.ShapeDtypeStruct((NCORES, NP, Wd), jnp.float32),
        mesh=_MESH,
        compiler_params=pltpu.CompilerParams(use_tc_tiling_on_sc=tc_tiling),
        scratch_types=[
            pltpu.VMEM((NJ1, CHUNK), jnp.int32),
            pltpu.VMEM((NJ1, CHUNK), jnp.int32),
            pltpu.VMEM((CHUNK, Wd), jnp.float32),
            pltpu.VMEM_SHARED((NP, Wd), jnp.float32),
            pltpu.SemaphoreType.DMA,
        ],
    )
    def _agg(g_hbm, row0_hbm, col0_hbm, row1_hbm, col1_hbm, out_hbm, row_v,
             col_v, stage_a, acc_sh, sem_a):
        c = lax.axis_index("c")
        s = lax.axis_index("s")
        t = c * NSUB + s
        kw = Wd // 16

        def fz(i, _):
            stage_a[i // kw, pl.ds((i % kw) * 16, 16)] = jnp.zeros((16,), jnp.float32)
            return 0

        lax.fori_loop(0, CHUNK * kw, fz, 0)

        def zc(k, _):
            pltpu.sync_copy(stage_a, acc_sh.at[pl.ds(s * NPT + k * CHUNK, CHUNK)])
            return 0

        lax.fori_loop(0, NPT // CHUNK, zc, 0)
        plsc.subcore_barrier()

        def run(row_hbm, col_hbm, njc):
            pltpu.sync_copy(row_hbm.at[s], row_v.at[pl.ds(0, njc)])
            pltpu.sync_copy(col_hbm.at[s], col_v.at[pl.ds(0, njc)])

            def body(j, _):
                pltpu.async_copy(g_hbm.at[row_v.at[j]], stage_a, sem_a).wait()
                pltpu.sync_copy(stage_a, acc_sh.at[col_v.at[j]], add=True)
                return 0

            lax.fori_loop(0, njc, body, 0)

        @pl.when(c == 0)
        def _():
            run(row0_hbm, col0_hbm, NJ0)

        @pl.when(c == 1)
        def _():
            run(row1_hbm, col1_hbm, NJ1)

        plsc.subcore_barrier()
        pltpu.sync_copy(acc_sh.at[pl.ds(s * NPT, NPT)],
                        out_hbm.at[c, pl.ds(s * NPT, NPT)])

    return _agg


_agg128 = _make_agg(H)

_R = 1000


def _tc1_body(hist_ref, x_ref, w_ref, g_ref, dis_ref):
    hist = hist_ref[...]
    deg = hist[0] + hist[1]
    dis = lax.rsqrt(deg[:, 0:1] + 1.0)
    g_ref[...] = jnp.dot(x_ref[...] * dis, w_ref[...],
                         preferred_element_type=jnp.float32)
    dis_ref[...] = dis


_tc1 = pl.pallas_call(
    _tc1_body,
    grid=(N // _R,),
    in_specs=[
        pl.BlockSpec((NCORES, _R, 16), lambda i: (0, i, 0)),
        pl.BlockSpec((_R, D), lambda i: (i, 0)),
        pl.BlockSpec((D, H), lambda i: (0, 0)),
    ],
    out_specs=[
        pl.BlockSpec((_R, H), lambda i: (i, 0)),
        pl.BlockSpec((_R, 1), lambda i: (i, 0)),
    ],
    out_shape=[
        jax.ShapeDtypeStruct((N, H), jnp.float32),
        jax.ShapeDtypeStruct((N, 1), jnp.float32),
    ],
)


def _tc2_body(acc_ref, g1_ref, dis_ref, b1_ref, w2_ref, hid_ref, g2_ref):
    a = acc_ref[0] + acc_ref[1] + g1_ref[...]
    dis = dis_ref[...]
    h = jnp.maximum(dis * a + b1_ref[...], 0.0)
    hid_ref[...] = h
    g2_ref[...] = dis * jnp.dot(h, w2_ref[...], preferred_element_type=jnp.float32)


_tc2 = pl.pallas_call(
    _tc2_body,
    grid=(N // _R,),
    in_specs=[
        pl.BlockSpec((NCORES, _R, H), lambda i: (0, i, 0)),
        pl.BlockSpec((_R, H), lambda i: (i, 0)),
        pl.BlockSpec((_R, 1), lambda i: (i, 0)),
        pl.BlockSpec((1, H), lambda i: (0, 0)),
        pl.BlockSpec((H, C_PAD), lambda i: (0, 0)),
    ],
    out_specs=[
        pl.BlockSpec((_R, H), lambda i: (i, 0)),
        pl.BlockSpec((_R, C_PAD), lambda i: (i, 0)),
    ],
    out_shape=[
        jax.ShapeDtypeStruct((N, H), jnp.float32),
        jax.ShapeDtypeStruct((N, C_PAD), jnp.float32),
    ],
)


def _tc3_body(acc_ref, g2_ref, dis_ref, b2_ref, out_ref):
    a = acc_ref[0] + acc_ref[1] + g2_ref[...]
    out_ref[...] = dis_ref[...] * a + b2_ref[...]


_tc3 = pl.pallas_call(
    _tc3_body,
    grid=(N // _R,),
    in_specs=[
        pl.BlockSpec((NCORES, _R, C_PAD), lambda i: (0, i, 0)),
        pl.BlockSpec((_R, C_PAD), lambda i: (i, 0)),
        pl.BlockSpec((_R, 1), lambda i: (i, 0)),
        pl.BlockSpec((1, C_PAD), lambda i: (0, 0)),
    ],
    out_specs=pl.BlockSpec((_R, C_PAD), lambda i: (i, 0)),
    out_shape=jax.ShapeDtypeStruct((N, C_PAD), jnp.float32),
)


def kernel(x, edge_index, W1, b1, W2, b2):
    row = edge_index[0]
    col = edge_index[1]
    pad = E_PAD - E
    rowp = jnp.concatenate([row, jnp.zeros((pad,), row.dtype)]).reshape(NCH, CHUNK)
    colp = jnp.concatenate([col, jnp.full((pad,), N, col.dtype)]).reshape(NCH, CHUNK)

    split = NSUB * NJ0
    row0 = rowp[:split].reshape(NSUB, NJ0, CHUNK)
    col0 = colp[:split].reshape(NSUB, NJ0, CHUNK)
    row1 = rowp[split:].reshape(NSUB, NJ1, CHUNK)
    col1 = colp[split:].reshape(NSUB, NJ1, CHUNK)

    hist = _hist_kernel(colp.reshape(NW, NJ, CHUNK))
    g1, dis = _tc1(hist, x, W1)
    acc1 = _agg128(g1, row0, col0, row1, col1)
    hidden, g2 = _tc2(acc1, g1, dis, b1.reshape(1, H),
                      jnp.zeros((H, C_PAD), W2.dtype).at[:, :C].set(W2))
    acc2 = _agg128(g2, row0, col0, row1, col1)
    out = _tc3(acc2, g2, dis, jnp.zeros((1, C_PAD), b2.dtype).at[0, :C].set(b2))
    return out[:, :C], hidden


# speedup vs baseline: 1.1790x; 1.1790x over previous
import functools

import jax
import jax.numpy as jnp
from jax import lax
from jax.experimental import pallas as pl
from jax.experimental.pallas import tpu as pltpu
from jax.experimental.pallas import tpu_sc as plsc

N = 10000
D = 128
H = 128
C = 40
E = 320000

NCORES = 2
NSUB = 16
NW = NCORES * NSUB
CHUNK = 128
NJ = 80
E_PAD = NW * CHUNK * NJ
NCH = E_PAD // CHUNK
NJ0 = 80
NJ1 = 80
NP = 10240
NPT = NP // NSUB
C_PAD = 128

_MESH = plsc.VectorSubcoreMesh(core_axis_name="c", subcore_axis_name="s")


@functools.partial(
    pl.kernel,
    out_type=jax.ShapeDtypeStruct((NCORES, NP, 16), jnp.float32),
    mesh=_MESH,
    scratch_types=[
        pltpu.VMEM((NJ, CHUNK), jnp.int32),
        pltpu.VMEM((CHUNK, 16), jnp.float32),
        pltpu.VMEM((CHUNK, 16), jnp.float32),
        pltpu.VMEM_SHARED((NP, 16), jnp.float32),
    ],
)
def _hist_kernel(col_hbm, hist_hbm, col_v, ones_v, zeros_v, hist_sh):
    c = lax.axis_index("c")
    s = lax.axis_index("s")
    t = c * NSUB + s

    def fill(i, _):
        ones_v[i, :] = jnp.ones((16,), jnp.float32)
        zeros_v[i, :] = jnp.zeros((16,), jnp.float32)
        return 0

    lax.fori_loop(0, CHUNK, fill, 0)

    def zc(k, _):
        pltpu.sync_copy(zeros_v, hist_sh.at[pl.ds(s * NPT + k * CHUNK, CHUNK)])
        return 0

    lax.fori_loop(0, NPT // CHUNK, zc, 0)
    pltpu.sync_copy(col_hbm.at[t], col_v)
    plsc.subcore_barrier()

    def body(j, _):
        pltpu.sync_copy(ones_v, hist_sh.at[col_v.at[j]], add=True)
        return 0

    lax.fori_loop(0, NJ, body, 0)
    plsc.subcore_barrier()
    pltpu.sync_copy(hist_sh.at[pl.ds(s * NPT, NPT)],
                    hist_hbm.at[c, pl.ds(s * NPT, NPT)])


def _make_agg(Wd, tc_tiling=True):
    @functools.partial(
        pl.kernel,
        out_type=jax.ShapeDtypeStruct((NCORES, NP, Wd), jnp.float32),
        mesh=_MESH,
        compiler_params=pltpu.CompilerParams(use_tc_tiling_on_sc=tc_tiling),
        scratch_types=[
            pltpu.VMEM((NJ, CHUNK), jnp.int32),
            pltpu.VMEM((NJ, CHUNK), jnp.int32),
            pltpu.VMEM((CHUNK, Wd), jnp.float32),
            pltpu.VMEM_SHARED((NP, Wd), jnp.float32),
            pltpu.SemaphoreType.DMA,
        ],
    )
    def _agg(g_hbm, row_hbm, col_hbm, out_hbm, row_v, col_v, stage_a,
             acc_sh, sem_a):
        c = lax.axis_index("c")
        s = lax.axis_index("s")
        t = c * NSUB + s
        kw = Wd // 16

        def fz(i, _):
            stage_a[i // kw, pl.ds((i % kw) * 16, 16)] = jnp.zeros((16,), jnp.float32)
            return 0

        lax.fori_loop(0, CHUNK * kw, fz, 0)

        def zc(k, _):
            pltpu.sync_copy(stage_a, acc_sh.at[pl.ds(s * NPT + k * CHUNK, CHUNK)])
            return 0

        lax.fori_loop(0, NPT // CHUNK, zc, 0)
        pltpu.sync_copy(row_hbm.at[t], row_v)
        pltpu.sync_copy(col_hbm.at[t], col_v)
        plsc.subcore_barrier()

        def body(j, _):
            pltpu.async_copy(g_hbm.at[row_v.at[j]], stage_a, sem_a).wait()
            pltpu.sync_copy(stage_a, acc_sh.at[col_v.at[j]], add=True)
            return 0

        lax.fori_loop(0, NJ, body, 0)
        plsc.subcore_barrier()
        pltpu.sync_copy(acc_sh.at[pl.ds(s * NPT, NPT)],
                        out_hbm.at[c, pl.ds(s * NPT, NPT)])

    return _agg


_agg128 = _make_agg(H)

_R = 1000


def _tc1_body(hist_ref, x_ref, w_ref, g_ref, dis_ref):
    hist = hist_ref[...]
    deg = hist[0] + hist[1]
    dis = lax.rsqrt(deg[:, 0:1] + 1.0)
    g_ref[...] = jnp.dot(x_ref[...] * dis, w_ref[...],
                         preferred_element_type=jnp.float32)
    dis_ref[...] = dis


_tc1 = pl.pallas_call(
    _tc1_body,
    grid=(N // _R,),
    in_specs=[
        pl.BlockSpec((NCORES, _R, 16), lambda i: (0, i, 0)),
        pl.BlockSpec((_R, D), lambda i: (i, 0)),
        pl.BlockSpec((D, H), lambda i: (0, 0)),
    ],
    out_specs=[
        pl.BlockSpec((_R, H), lambda i: (i, 0)),
        pl.BlockSpec((_R, 1), lambda i: (i, 0)),
    ],
    out_shape=[
        jax.ShapeDtypeStruct((N, H), jnp.float32),
        jax.ShapeDtypeStruct((N, 1), jnp.float32),
    ],
)


def _tc2_body(acc_ref, g1_ref, dis_ref, b1_ref, w2_ref, hid_ref, g2_ref):
    a = acc_ref[0] + acc_ref[1] + g1_ref[...]
    dis = dis_ref[...]
    h = jnp.maximum(dis * a + b1_ref[...], 0.0)
    hid_ref[...] = h
    g2_ref[...] = dis * jnp.dot(h, w2_ref[...], preferred_element_type=jnp.float32)


_tc2 = pl.pallas_call(
    _tc2_body,
    grid=(N // _R,),
    in_specs=[
        pl.BlockSpec((NCORES, _R, H), lambda i: (0, i, 0)),
        pl.BlockSpec((_R, H), lambda i: (i, 0)),
        pl.BlockSpec((_R, 1), lambda i: (i, 0)),
        pl.BlockSpec((1, H), lambda i: (0, 0)),
        pl.BlockSpec((H, C_PAD), lambda i: (0, 0)),
    ],
    out_specs=[
        pl.BlockSpec((_R, H), lambda i: (i, 0)),
        pl.BlockSpec((_R, C_PAD), lambda i: (i, 0)),
    ],
    out_shape=[
        jax.ShapeDtypeStruct((N, H), jnp.float32),
        jax.ShapeDtypeStruct((N, C_PAD), jnp.float32),
    ],
)


def _tc3_body(acc_ref, g2_ref, dis_ref, b2_ref, out_ref):
    a = acc_ref[0] + acc_ref[1] + g2_ref[...]
    out_ref[...] = dis_ref[...] * a + b2_ref[...]


_tc3 = pl.pallas_call(
    _tc3_body,
    grid=(N // _R,),
    in_specs=[
        pl.BlockSpec((NCORES, _R, C_PAD), lambda i: (0, i, 0)),
        pl.BlockSpec((_R, C_PAD), lambda i: (i, 0)),
        pl.BlockSpec((_R, 1), lambda i: (i, 0)),
        pl.BlockSpec((1, C_PAD), lambda i: (0, 0)),
    ],
    out_specs=pl.BlockSpec((_R, C_PAD), lambda i: (i, 0)),
    out_shape=jax.ShapeDtypeStruct((N, C_PAD), jnp.float32),
)


def kernel(x, edge_index, W1, b1, W2, b2):
    row = edge_index[0]
    col = edge_index[1]
    pad = E_PAD - E
    rowp = jnp.concatenate([row, jnp.zeros((pad,), row.dtype)]).reshape(NCH, CHUNK)
    colp = jnp.concatenate([col, jnp.full((pad,), N, col.dtype)]).reshape(NCH, CHUNK)

    rowp3 = rowp.reshape(NW, NJ, CHUNK)
    colp3 = colp.reshape(NW, NJ, CHUNK)

    hist = _hist_kernel(colp3)
    g1, dis = _tc1(hist, x, W1)
    acc1 = _agg128(g1, rowp3, colp3)
    hidden, g2 = _tc2(acc1, g1, dis, b1.reshape(1, H),
                      jnp.zeros((H, C_PAD), W2.dtype).at[:, :C].set(W2))
    acc2 = _agg128(g2, rowp3, colp3)
    out = _tc3(acc2, g2, dis, jnp.zeros((1, C_PAD), b2.dtype).at[0, :C].set(b2))
    return out[:, :C], hidden


# speedup vs baseline: 1.6917x; 1.4348x over previous
import functools

import jax
import jax.numpy as jnp
from jax import lax
from jax.experimental import pallas as pl
from jax.experimental.pallas import tpu as pltpu
from jax.experimental.pallas import tpu_sc as plsc

N = 10000
D = 128
H = 128
C = 40
E = 320000

NCORES = 2
NSUB = 16
NW = NCORES * NSUB
CHUNK = 128
NJ = -(-E // (NW * CHUNK))
E_PAD = NW * CHUNK * NJ
NP = 10240
NPT = NP // NSUB
C_PAD = 128

_MESH = plsc.VectorSubcoreMesh(core_axis_name="c", subcore_axis_name="s")


@functools.partial(
    pl.kernel,
    out_type=jax.ShapeDtypeStruct((NCORES, NP, 16), jnp.float32),
    mesh=_MESH,
    scratch_types=[
        pltpu.VMEM((NJ, CHUNK), jnp.int32),
        pltpu.VMEM((CHUNK, 16), jnp.float32),
        pltpu.VMEM((CHUNK, 16), jnp.float32),
        pltpu.VMEM_SHARED((NP, 16), jnp.float32),
    ],
)
def _hist_kernel(col_hbm, hist_hbm, col_v, ones_v, zeros_v, hist_sh):
    c = lax.axis_index("c")
    s = lax.axis_index("s")
    t = c * NSUB + s

    def fill(i, _):
        ones_v[i, :] = jnp.ones((16,), jnp.float32)
        zeros_v[i, :] = jnp.zeros((16,), jnp.float32)
        return 0

    lax.fori_loop(0, CHUNK, fill, 0)

    def zc(k, _):
        pltpu.sync_copy(zeros_v, hist_sh.at[pl.ds(s * NPT + k * CHUNK, CHUNK)])
        return 0

    lax.fori_loop(0, NPT // CHUNK, zc, 0)
    pltpu.sync_copy(col_hbm.at[t], col_v)
    plsc.subcore_barrier()

    def body(j, _):
        pltpu.sync_copy(ones_v, hist_sh.at[col_v.at[j]], add=True)
        return 0

    lax.fori_loop(0, NJ, body, 0)
    plsc.subcore_barrier()

    def wb(k, _):
        pltpu.sync_copy(hist_sh.at[pl.ds(s * NPT + k * CHUNK, CHUNK)],
                        hist_hbm.at[c, pl.ds(s * NPT + k * CHUNK, CHUNK)])
        return 0

    lax.fori_loop(0, NPT // CHUNK, wb, 0)


def _make_agg(Wd):
    @functools.partial(
        pl.kernel,
        out_type=jax.ShapeDtypeStruct((NCORES, NP, Wd), jnp.float32),
        mesh=_MESH,
        scratch_types=[
            pltpu.VMEM((NJ, CHUNK), jnp.int32),
            pltpu.VMEM((NJ, CHUNK), jnp.int32),
            pltpu.VMEM((CHUNK, Wd), jnp.float32),
            pltpu.VMEM_SHARED((NP, Wd), jnp.float32),
            pltpu.SemaphoreType.DMA,
        ],
    )
    def _agg(g_hbm, row_hbm, col_hbm, out_hbm, row_v, col_v, stage_v,
             acc_sh, sem):
        c = lax.axis_index("c")
        s = lax.axis_index("s")
        t = c * NSUB + s
        kw = Wd // 16

        def fz(i, _):
            stage_v[i // kw, pl.ds((i % kw) * 16, 16)] = jnp.zeros((16,), jnp.float32)
            return 0

        lax.fori_loop(0, CHUNK * kw, fz, 0)

        def zc(k, _):
            pltpu.sync_copy(stage_v, acc_sh.at[pl.ds(s * NPT + k * CHUNK, CHUNK)])
            return 0

        lax.fori_loop(0, NPT // CHUNK, zc, 0)
        pltpu.sync_copy(row_hbm.at[t], row_v)
        pltpu.sync_copy(col_hbm.at[t], col_v)
        plsc.subcore_barrier()

        def body(j, _):
            pltpu.async_copy(g_hbm.at[row_v.at[j]], stage_v, sem).wait()
            pltpu.sync_copy(stage_v, acc_sh.at[col_v.at[j]], add=True)
            return 0

        lax.fori_loop(0, NJ, body, 0)
        plsc.subcore_barrier()

        def wb(k, _):
            pltpu.sync_copy(acc_sh.at[pl.ds(s * NPT + k * CHUNK, CHUNK)],
                            out_hbm.at[c, pl.ds(s * NPT + k * CHUNK, CHUNK)])
            return 0

        lax.fori_loop(0, NPT // CHUNK, wb, 0)

    return _agg


_agg128 = _make_agg(H)

_R = 1000


def _tc1_body(hist_ref, x_ref, w_ref, g_ref, dis_ref):
    hist = hist_ref[...]
    deg = hist[0] + hist[1]
    dis = lax.rsqrt(deg[:, 0:1] + 1.0)
    g_ref[...] = jnp.dot(x_ref[...] * dis, w_ref[...],
                         preferred_element_type=jnp.float32)
    dis_ref[...] = dis


_tc1 = pl.pallas_call(
    _tc1_body,
    grid=(N // _R,),
    in_specs=[
        pl.BlockSpec((NCORES, _R, 16), lambda i: (0, i, 0)),
        pl.BlockSpec((_R, D), lambda i: (i, 0)),
        pl.BlockSpec((D, H), lambda i: (0, 0)),
    ],
    out_specs=[
        pl.BlockSpec((_R, H), lambda i: (i, 0)),
        pl.BlockSpec((_R, 1), lambda i: (i, 0)),
    ],
    out_shape=[
        jax.ShapeDtypeStruct((N, H), jnp.float32),
        jax.ShapeDtypeStruct((N, 1), jnp.float32),
    ],
)


def _tc2_body(acc_ref, g1_ref, dis_ref, b1_ref, w2_ref, hid_ref, g2_ref):
    a = acc_ref[0] + acc_ref[1] + g1_ref[...]
    dis = dis_ref[...]
    h = jnp.maximum(dis * a + b1_ref[...], 0.0)
    hid_ref[...] = h
    g2_ref[...] = dis * jnp.dot(h, w2_ref[...], preferred_element_type=jnp.float32)


_tc2 = pl.pallas_call(
    _tc2_body,
    grid=(N // _R,),
    in_specs=[
        pl.BlockSpec((NCORES, _R, H), lambda i: (0, i, 0)),
        pl.BlockSpec((_R, H), lambda i: (i, 0)),
        pl.BlockSpec((_R, 1), lambda i: (i, 0)),
        pl.BlockSpec((1, H), lambda i: (0, 0)),
        pl.BlockSpec((H, C_PAD), lambda i: (0, 0)),
    ],
    out_specs=[
        pl.BlockSpec((_R, H), lambda i: (i, 0)),
        pl.BlockSpec((_R, C_PAD), lambda i: (i, 0)),
    ],
    out_shape=[
        jax.ShapeDtypeStruct((N, H), jnp.float32),
        jax.ShapeDtypeStruct((N, C_PAD), jnp.float32),
    ],
)


def _tc3_body(acc_ref, g2_ref, dis_ref, b2_ref, out_ref):
    a = acc_ref[0] + acc_ref[1] + g2_ref[...]
    out_ref[...] = dis_ref[...] * a + b2_ref[...]


_tc3 = pl.pallas_call(
    _tc3_body,
    grid=(N // _R,),
    in_specs=[
        pl.BlockSpec((NCORES, _R, C_PAD), lambda i: (0, i, 0)),
        pl.BlockSpec((_R, C_PAD), lambda i: (i, 0)),
        pl.BlockSpec((_R, 1), lambda i: (i, 0)),
        pl.BlockSpec((1, C_PAD), lambda i: (0, 0)),
    ],
    out_specs=pl.BlockSpec((_R, C_PAD), lambda i: (i, 0)),
    out_shape=jax.ShapeDtypeStruct((N, C_PAD), jnp.float32),
)


def kernel(x, edge_index, W1, b1, W2, b2):
    row = edge_index[0]
    col = edge_index[1]
    pad = E_PAD - E
    rowp = jnp.concatenate([row, jnp.zeros((pad,), row.dtype)]).reshape(NW, NJ, CHUNK)
    colp = jnp.concatenate([col, jnp.full((pad,), N, col.dtype)]).reshape(NW, NJ, CHUNK)

    hist = _hist_kernel(colp)[:, :N, :]
    g1, dis = _tc1(hist, x, W1)
    acc1 = _agg128(g1, rowp, colp)[:, :N, :]
    hidden, g2 = _tc2(acc1, g1, dis, b1.reshape(1, H),
                      jnp.zeros((H, C_PAD), W2.dtype).at[:, :C].set(W2))
    acc2 = _agg128(g2, rowp, colp)[:, :N, :]
    out = _tc3(acc2, g2, dis, jnp.zeros((1, C_PAD), b2.dtype).at[0, :C].set(b2))
    return out[:, :C], hidden
